# Initial kernel scaffold; baseline (speedup 1.0000x reference)
import jax
import jax.numpy as jnp
from jax.experimental import pallas as pl


def kernel(x, edge_index, w2b, W1, b1, W2, b2, W3, b3, LW1, LB1, LW2, LB2):
    raise NotImplementedError("write your pallas kernel here")


# speedup vs baseline: 11.5658x; 11.5658x over previous
import functools

import jax
import jax.numpy as jnp
from jax import lax
from jax.experimental import pallas as pl
from jax.experimental.pallas import tpu as pltpu
from jax.experimental.pallas import tpu_sc as plsc

_LANES = 128
_KS = 16
_BN = 3584
_BE = 6400


def _sc_mesh():
    return plsc.VectorSubcoreMesh(core_axis_name="c", subcore_axis_name="s")


def _deg_call(dst_rows, zeros1):
    npad = zeros1.shape[0]
    epr = dst_rows.shape[0]
    per_sc = epr // 2
    per_tile = per_sc // 16
    outer = per_tile // _KS
    rows_out = npad // 16

    @functools.partial(
        pl.kernel,
        out_type=jax.ShapeDtypeStruct((2 * npad,), jnp.float32),
        mesh=_sc_mesh(),
        compiler_params=pltpu.CompilerParams(use_tc_tiling_on_sc=False),
        scratch_types=[
            pltpu.VMEM((_KS, _LANES), jnp.int32),
            pltpu.VMEM((_LANES,), jnp.float32),
            pltpu.VMEM_SHARED((npad,), jnp.float32),
        ],
    )
    def k(dst_hbm, z_hbm, out_hbm, idx_v, ones_v, acc):
        c = lax.axis_index("c")
        s = lax.axis_index("s")
        for i in range(_LANES // 16):
            ones_v[pl.ds(16 * i, 16)] = jnp.ones((16,), jnp.float32)
        pltpu.sync_copy(z_hbm.at[pl.ds(s * rows_out, rows_out)],
                        acc.at[pl.ds(s * rows_out, rows_out)])
        plsc.subcore_barrier()
        base = c * per_sc + s * per_tile

        def outer_body(o, carry):
            pltpu.sync_copy(dst_hbm.at[pl.ds(base + o * _KS, _KS)], idx_v)

            def inner(j, cc):
                pltpu.sync_copy(ones_v, acc.at[idx_v.at[j]], add=True)
                return cc

            return lax.fori_loop(0, _KS, inner, carry)

        lax.fori_loop(0, outer, outer_body, 0)
        plsc.subcore_barrier()
        pltpu.sync_copy(acc.at[pl.ds(s * rows_out, rows_out)],
                        out_hbm.at[pl.ds(c * npad + s * rows_out, rows_out)])

    return k(dst_rows, zeros1)


def _agg_edge_call(g, src_rows, dst_rows, zeros_d):
    npad, d = g.shape
    epr = src_rows.shape[0]
    per_sc = epr // 2
    per_tile = per_sc // 16
    outer = per_tile // _KS
    rows_out = npad // 16

    @functools.partial(
        pl.kernel,
        out_type=jax.ShapeDtypeStruct((2, npad, d), jnp.float32),
        mesh=_sc_mesh(),
        compiler_params=pltpu.CompilerParams(use_tc_tiling_on_sc=False),
        scratch_types=[
            pltpu.VMEM((_KS, _LANES), jnp.int32),
            pltpu.VMEM((_KS, _LANES), jnp.int32),
            pltpu.VMEM((_LANES, d), jnp.float32),
            pltpu.VMEM_SHARED((npad, d), jnp.float32),
            pltpu.SemaphoreType.DMA,
        ],
    )
    def k(g_hbm, src_hbm, dst_hbm, z_hbm, out_hbm,
          idxs_v, idxd_v, rows_v, acc, sem):
        c = lax.axis_index("c")
        s = lax.axis_index("s")
        pltpu.sync_copy(z_hbm.at[pl.ds(s * rows_out, rows_out)],
                        acc.at[pl.ds(s * rows_out, rows_out)])
        plsc.subcore_barrier()
        base = c * per_sc + s * per_tile

        def outer_body(o, carry):
            pltpu.sync_copy(src_hbm.at[pl.ds(base + o * _KS, _KS)], idxs_v)
            pltpu.sync_copy(dst_hbm.at[pl.ds(base + o * _KS, _KS)], idxd_v)

            def inner(j, cc):
                pltpu.async_copy(g_hbm.at[idxs_v.at[j]], rows_v, sem).wait()
                pltpu.sync_copy(rows_v, acc.at[idxd_v.at[j]], add=True)
                return cc

            return lax.fori_loop(0, _KS, inner, carry)

        lax.fori_loop(0, outer, outer_body, 0)
        plsc.subcore_barrier()
        pltpu.sync_copy(acc.at[pl.ds(s * rows_out, rows_out)],
                        out_hbm.at[c, pl.ds(s * rows_out, rows_out)])

    return k(g, src_rows, dst_rows, zeros_d)


def _agg_col_call(ga, gb, src_rows, dst_rows, zeros_d):
    npad, d = ga.shape
    epr = src_rows.shape[0]
    per_tile = epr // 16
    outer = per_tile // _KS
    rows_out = npad // 16

    @functools.partial(
        pl.kernel,
        out_type=jax.ShapeDtypeStruct((2, npad, d), jnp.float32),
        mesh=_sc_mesh(),
        compiler_params=pltpu.CompilerParams(use_tc_tiling_on_sc=False),
        scratch_types=[
            pltpu.VMEM((_KS, _LANES), jnp.int32),
            pltpu.VMEM((_KS, _LANES), jnp.int32),
            pltpu.VMEM((_LANES, d), jnp.float32),
            pltpu.VMEM_SHARED((npad, d), jnp.float32),
            pltpu.SemaphoreType.DMA,
        ],
    )
    def k(ga_hbm, gb_hbm, src_hbm, dst_hbm, z_hbm, out_hbm,
          idxs_v, idxd_v, rows_v, acc, sem):
        c = lax.axis_index("c")
        s = lax.axis_index("s")
        pltpu.sync_copy(z_hbm.at[pl.ds(s * rows_out, rows_out)],
                        acc.at[pl.ds(s * rows_out, rows_out)])
        plsc.subcore_barrier()
        base = s * per_tile

        def run(src_g):
            def outer_body(o, carry):
                pltpu.sync_copy(src_hbm.at[pl.ds(base + o * _KS, _KS)], idxs_v)
                pltpu.sync_copy(dst_hbm.at[pl.ds(base + o * _KS, _KS)], idxd_v)

                def inner(j, cc):
                    pltpu.async_copy(src_g.at[idxs_v.at[j]], rows_v, sem).wait()
                    pltpu.sync_copy(rows_v, acc.at[idxd_v.at[j]], add=True)
                    return cc

                return lax.fori_loop(0, _KS, inner, carry)

            lax.fori_loop(0, outer, outer_body, 0)

        @pl.when(c == 0)
        def _():
            run(ga_hbm)

        @pl.when(c != 0)
        def _():
            run(gb_hbm)

        plsc.subcore_barrier()
        pltpu.sync_copy(acc.at[pl.ds(s * rows_out, rows_out)],
                        out_hbm.at[c, pl.ds(s * rows_out, rows_out)])

    return k(ga, gb, src_rows, dst_rows, zeros_d)


def _link_gather_call(h3, idx_rows):
    npad, d = h3.shape
    lpr = idx_rows.shape[0]
    per_tile = lpr // 32
    outer = per_tile // _KS

    @functools.partial(
        pl.kernel,
        out_type=jax.ShapeDtypeStruct((lpr * _LANES, d), jnp.float32),
        mesh=_sc_mesh(),
        compiler_params=pltpu.CompilerParams(use_tc_tiling_on_sc=False),
        scratch_types=[
            pltpu.VMEM((_KS, _LANES), jnp.int32),
            pltpu.VMEM((_LANES, d), jnp.float32),
            pltpu.SemaphoreType.DMA,
        ],
    )
    def k(h_hbm, idx_hbm, out_hbm, idx_v, rows_v, sem):
        c = lax.axis_index("c")
        s = lax.axis_index("s")
        base = (s * 2 + c) * per_tile

        def outer_body(o, carry):
            pltpu.sync_copy(idx_hbm.at[pl.ds(base + o * _KS, _KS)], idx_v)

            def inner(j, cc):
                pltpu.async_copy(h_hbm.at[idx_v.at[j]], rows_v, sem).wait()
                pltpu.sync_copy(
                    rows_v,
                    out_hbm.at[pl.ds((base + o * _KS + j) * _LANES, _LANES)])
                return cc

            return lax.fori_loop(0, _KS, inner, carry)

        lax.fori_loop(0, outer, outer_body, 0)

    return k(h3, idx_rows)


def _tc_first_call(deg2, xp, w1):
    npad, f = xp.shape
    dn = w1.shape[1]
    grid = npad // _BN

    def body(d_ref, x_ref, w_ref, dinv_ref, g_ref):
        deg = d_ref[0] + d_ref[1] + 1.0
        dinv = lax.rsqrt(deg)
        h = jnp.dot(x_ref[...], w_ref[...], preferred_element_type=jnp.float32)
        dinv_ref[...] = dinv[:, None]
        g_ref[...] = h * dinv[:, None]

    return pl.pallas_call(
        body,
        grid=(grid,),
        in_specs=[pl.BlockSpec((2, _BN), lambda i: (0, i)),
                  pl.BlockSpec((_BN, f), lambda i: (i, 0)),
                  pl.BlockSpec(w1.shape, lambda i: (0, 0))],
        out_specs=[pl.BlockSpec((_BN, 1), lambda i: (i, 0)),
                   pl.BlockSpec((_BN, dn), lambda i: (i, 0))],
        out_shape=[jax.ShapeDtypeStruct((npad, 1), jnp.float32),
                   jax.ShapeDtypeStruct((npad, dn), jnp.float32)],
    )(deg2, xp, w1)


def _tc_layer_call(acc, gprev, dinv, b, w, split):
    _, npad, di = acc.shape
    dn = w.shape[1]
    grid = npad // _BN

    def body(a_ref, g_ref, di_ref, b_ref, w_ref, *outs):
        dv = di_ref[...]
        h = jax.nn.relu(dv * (a_ref[0] + a_ref[1] + g_ref[...])
                        + b_ref[...])
        gn = jnp.dot(h, w_ref[...],
                     preferred_element_type=jnp.float32) * dv
        if split:
            outs[0][...] = gn[:, :dn // 2]
            outs[1][...] = gn[:, dn // 2:]
        else:
            outs[0][...] = gn

    if split:
        out_specs = [pl.BlockSpec((_BN, dn // 2), lambda i: (i, 0)),
                     pl.BlockSpec((_BN, dn // 2), lambda i: (i, 0))]
        out_shape = [jax.ShapeDtypeStruct((npad, dn // 2), jnp.float32),
                     jax.ShapeDtypeStruct((npad, dn // 2), jnp.float32)]
    else:
        out_specs = [pl.BlockSpec((_BN, dn), lambda i: (i, 0))]
        out_shape = [jax.ShapeDtypeStruct((npad, dn), jnp.float32)]

    return pl.pallas_call(
        body,
        grid=(grid,),
        in_specs=[pl.BlockSpec((2, _BN, di), lambda i: (0, i, 0)),
                  pl.BlockSpec((_BN, di), lambda i: (i, 0)),
                  pl.BlockSpec((_BN, 1), lambda i: (i, 0)),
                  pl.BlockSpec(b.shape, lambda i: (0,)),
                  pl.BlockSpec(w.shape, lambda i: (0, 0))],
        out_specs=out_specs,
        out_shape=out_shape,
    )(acc, gprev, dinv, b, w)


def _tc_h3_call(acc3, g3a, g3b, dinv, b3):
    _, npad, dh = acc3.shape
    grid = npad // _BN

    def body(a_ref, ga_ref, gb_ref, di_ref, b_ref, h_ref):
        dv = di_ref[...]
        bb = b_ref[...]
        ha = jax.nn.relu(dv * (a_ref[0] + ga_ref[...]) + bb[:dh])
        hb = jax.nn.relu(dv * (a_ref[1] + gb_ref[...]) + bb[dh:])
        h_ref[...] = jnp.concatenate([ha, hb], axis=1)

    return pl.pallas_call(
        body,
        grid=(grid,),
        in_specs=[pl.BlockSpec((2, _BN, dh), lambda i: (0, i, 0)),
                  pl.BlockSpec((_BN, dh), lambda i: (i, 0)),
                  pl.BlockSpec((_BN, dh), lambda i: (i, 0)),
                  pl.BlockSpec((_BN, 1), lambda i: (i, 0)),
                  pl.BlockSpec((2 * dh,), lambda i: (0,))],
        out_specs=pl.BlockSpec((_BN, 2 * dh), lambda i: (i, 0)),
        out_shape=jax.ShapeDtypeStruct((npad, 2 * dh), jnp.float32),
    )(acc3, g3a, g3b, dinv, b3)


def _tc_link_call(gall, lw1a, lw1b, lb1, lw2, lb2, ew):
    d = gall.shape[1]
    dh = lw1a.shape[1]
    do = lw2.shape[1]
    grid = ew // _BE

    def body(a_ref, b_ref, w1a, w1b, l1, w2, l2, o_ref):
        hidden = (jnp.dot(a_ref[...], w1a[...],
                          preferred_element_type=jnp.float32)
                  + jnp.dot(b_ref[...], w1b[...],
                            preferred_element_type=jnp.float32)
                  + l1[...])
        o_ref[...] = jnp.dot(hidden, w2[...],
                             preferred_element_type=jnp.float32) + l2[...]

    return pl.pallas_call(
        body,
        grid=(grid,),
        in_specs=[pl.BlockSpec((_BE, d), lambda i: (i, 0)),
                  pl.BlockSpec((_BE, d), lambda i, g=grid: (i + g, 0)),
                  pl.BlockSpec((d, dh), lambda i: (0, 0)),
                  pl.BlockSpec((d, dh), lambda i: (0, 0)),
                  pl.BlockSpec((dh,), lambda i: (0,)),
                  pl.BlockSpec((dh, do), lambda i: (0, 0)),
                  pl.BlockSpec((do,), lambda i: (0,))],
        out_specs=pl.BlockSpec((_BE, do), lambda i: (i, 0)),
        out_shape=jax.ShapeDtypeStruct((ew, do), jnp.float32),
    )(gall, gall, lw1a, lw1b, lb1, lw2, lb2)


def kernel(x, edge_index, w2b, W1, b1, W2, b2, W3, b3, LW1, LB1, LW2, LB2):
    n, _ = x.shape
    e = edge_index.shape[1]
    ew = w2b.shape[1]
    d3 = W3.shape[1]

    npad = ((n + 1 + _BN - 1) // _BN) * _BN
    eq = 2 * 16 * _KS * _LANES
    epad = ((e + eq - 1) // eq) * eq
    lq = 32 * _KS * _LANES
    lpad = ((2 * ew + lq - 1) // lq) * lq

    idx_pad = jnp.full((epad - e,), n, jnp.int32)
    srcp = jnp.concatenate([edge_index[0], idx_pad]).reshape(-1, _LANES)
    dstp = jnp.concatenate([edge_index[1], idx_pad]).reshape(-1, _LANES)
    lidx = jnp.concatenate(
        [w2b[0], w2b[1],
         jnp.full((lpad - 2 * ew,), n, jnp.int32)]).reshape(-1, _LANES)
    xp = jnp.pad(x, ((0, npad - n), (0, 0)))

    z1 = jnp.zeros((npad,), jnp.float32)
    zA = jnp.zeros((npad, W1.shape[1]), jnp.float32)
    zB = jnp.zeros((npad, W2.shape[1]), jnp.float32)

    deg2 = _deg_call(dstp, z1).reshape(2, npad)
    dinv, g1 = _tc_first_call(deg2, xp, W1)
    acc1 = _agg_edge_call(g1, srcp, dstp, zA)
    (g2,) = _tc_layer_call(acc1, g1, dinv, b1, W2, split=False)
    acc2 = _agg_edge_call(g2, srcp, dstp, zB)
    g3a, g3b = _tc_layer_call(acc2, g2, dinv, b2, W3, split=True)
    acc3 = _agg_col_call(g3a, g3b, srcp, dstp, zB)
    h3 = _tc_h3_call(acc3, g3a, g3b, dinv, b3)
    gall = _link_gather_call(h3, lidx)
    return _tc_link_call(gall, LW1[:d3], LW1[d3:], LB1, LW2, LB2, ew)


# speedup vs baseline: 17.7448x; 1.5342x over previous
import functools

import jax
import jax.numpy as jnp
from jax import lax
from jax.experimental import pallas as pl
from jax.experimental.pallas import tpu as pltpu
from jax.experimental.pallas import tpu_sc as plsc

_LANES = 128
_CP = 56
_BN = 3584
_BE = 6400


def _sc_mesh():
    return plsc.VectorSubcoreMesh(core_axis_name="c", subcore_axis_name="s")


def _deg_call(dst_rows, zeros1):
    npad = zeros1.shape[0]
    epr = dst_rows.shape[0]
    per_sc = epr // 2
    per_tile = per_sc // 16
    phases = per_tile // _CP
    assert per_tile == phases * _CP
    rows_out = npad // 16

    @functools.partial(
        pl.kernel,
        out_type=jax.ShapeDtypeStruct((2 * npad,), jnp.float32),
        mesh=_sc_mesh(),
        compiler_params=pltpu.CompilerParams(use_tc_tiling_on_sc=False),
        scratch_types=[
            pltpu.VMEM((_CP, _LANES), jnp.int32),
            pltpu.VMEM((_LANES,), jnp.float32),
            pltpu.VMEM_SHARED((npad,), jnp.float32),
        ],
    )
    def k(dst_hbm, z_hbm, out_hbm, idx_v, ones_v, acc):
        c = lax.axis_index("c")
        s = lax.axis_index("s")
        for i in range(_LANES // 16):
            ones_v[pl.ds(16 * i, 16)] = jnp.ones((16,), jnp.float32)
        pltpu.sync_copy(z_hbm.at[pl.ds(s * rows_out, rows_out)],
                        acc.at[pl.ds(s * rows_out, rows_out)])
        plsc.subcore_barrier()
        base = c * per_sc + s * per_tile
        for ph in range(phases):
            pltpu.sync_copy(dst_hbm.at[pl.ds(base + ph * _CP, _CP)], idx_v)

            def body(j, cc):
                pltpu.sync_copy(ones_v, acc.at[idx_v.at[j]], add=True)
                return cc

            lax.fori_loop(0, _CP, body, 0)
        plsc.subcore_barrier()
        pltpu.sync_copy(acc.at[pl.ds(s * rows_out, rows_out)],
                        out_hbm.at[pl.ds(c * npad + s * rows_out, rows_out)])

    return k(dst_rows, zeros1)


def _agg_pipeline(g_hbm, src_hbm, dst_hbm, acc, idxs_v, idxd_v,
                  rows0, rows1, sem0, sem1, base, phases):
    for ph in range(phases):
        pb = base + ph * _CP
        pltpu.sync_copy(src_hbm.at[pl.ds(pb, _CP)], idxs_v)
        pltpu.sync_copy(dst_hbm.at[pl.ds(pb, _CP)], idxd_v)
        pltpu.async_copy(g_hbm.at[idxs_v.at[0]], rows0, sem0)

        def pair(m, cc):
            j0 = 2 * m
            pltpu.async_copy(g_hbm.at[idxs_v.at[j0 + 1]], rows1, sem1)
            pltpu.make_async_copy(g_hbm.at[idxs_v.at[0]], rows0, sem0).wait()
            pltpu.sync_copy(rows0, acc.at[idxd_v.at[j0]], add=True)

            @pl.when(m < _CP // 2 - 1)
            def _():
                pltpu.async_copy(g_hbm.at[idxs_v.at[j0 + 2]], rows0, sem0)

            pltpu.make_async_copy(g_hbm.at[idxs_v.at[0]], rows1, sem1).wait()
            pltpu.sync_copy(rows1, acc.at[idxd_v.at[j0 + 1]], add=True)
            return cc

        lax.fori_loop(0, _CP // 2, pair, 0)


def _agg_edge_call(g, src_rows, dst_rows, zeros_d):
    npad, d = g.shape
    epr = src_rows.shape[0]
    per_sc = epr // 2
    per_tile = per_sc // 16
    phases = per_tile // _CP
    assert per_tile == phases * _CP
    rows_out = npad // 16

    @functools.partial(
        pl.kernel,
        out_type=jax.ShapeDtypeStruct((2, npad, d), jnp.float32),
        mesh=_sc_mesh(),
        compiler_params=pltpu.CompilerParams(use_tc_tiling_on_sc=False),
        scratch_types=[
            pltpu.VMEM((_CP, _LANES), jnp.int32),
            pltpu.VMEM((_CP, _LANES), jnp.int32),
            pltpu.VMEM((_LANES, d), jnp.float32),
            pltpu.VMEM((_LANES, d), jnp.float32),
            pltpu.VMEM_SHARED((npad, d), jnp.float32),
            pltpu.SemaphoreType.DMA,
            pltpu.SemaphoreType.DMA,
        ],
    )
    def k(g_hbm, src_hbm, dst_hbm, z_hbm, out_hbm,
          idxs_v, idxd_v, rows0, rows1, acc, sem0, sem1):
        c = lax.axis_index("c")
        s = lax.axis_index("s")
        pltpu.sync_copy(z_hbm.at[pl.ds(s * rows_out, rows_out)],
                        acc.at[pl.ds(s * rows_out, rows_out)])
        plsc.subcore_barrier()
        base = c * per_sc + s * per_tile
        _agg_pipeline(g_hbm, src_hbm, dst_hbm, acc, idxs_v, idxd_v,
                      rows0, rows1, sem0, sem1, base, phases=phases)
        plsc.subcore_barrier()
        pltpu.sync_copy(acc.at[pl.ds(s * rows_out, rows_out)],
                        out_hbm.at[c, pl.ds(s * rows_out, rows_out)])

    return k(g, src_rows, dst_rows, zeros_d)


def _agg_col_call(ga, gb, src_rows, dst_rows, zeros_d):
    npad, d = ga.shape
    epr = src_rows.shape[0]
    per_tile = epr // 16
    phases = per_tile // _CP
    assert per_tile == phases * _CP
    rows_out = npad // 16

    @functools.partial(
        pl.kernel,
        out_type=jax.ShapeDtypeStruct((2, npad, d), jnp.float32),
        mesh=_sc_mesh(),
        compiler_params=pltpu.CompilerParams(use_tc_tiling_on_sc=False),
        scratch_types=[
            pltpu.VMEM((_CP, _LANES), jnp.int32),
            pltpu.VMEM((_CP, _LANES), jnp.int32),
            pltpu.VMEM((_LANES, d), jnp.float32),
            pltpu.VMEM((_LANES, d), jnp.float32),
            pltpu.VMEM_SHARED((npad, d), jnp.float32),
            pltpu.SemaphoreType.DMA,
            pltpu.SemaphoreType.DMA,
        ],
    )
    def k(ga_hbm, gb_hbm, src_hbm, dst_hbm, z_hbm, out_hbm,
          idxs_v, idxd_v, rows0, rows1, acc, sem0, sem1):
        c = lax.axis_index("c")
        s = lax.axis_index("s")
        pltpu.sync_copy(z_hbm.at[pl.ds(s * rows_out, rows_out)],
                        acc.at[pl.ds(s * rows_out, rows_out)])
        plsc.subcore_barrier()
        base = s * per_tile

        @pl.when(c == 0)
        def _():
            _agg_pipeline(ga_hbm, src_hbm, dst_hbm, acc, idxs_v, idxd_v,
                          rows0, rows1, sem0, sem1, base, phases=phases)

        @pl.when(c != 0)
        def _():
            _agg_pipeline(gb_hbm, src_hbm, dst_hbm, acc, idxs_v, idxd_v,
                          rows0, rows1, sem0, sem1, base, phases=phases)

        plsc.subcore_barrier()
        pltpu.sync_copy(acc.at[pl.ds(s * rows_out, rows_out)],
                        out_hbm.at[c, pl.ds(s * rows_out, rows_out)])

    return k(ga, gb, src_rows, dst_rows, zeros_d)


def _link_gather_call(tbl, idx_rows):
    _, d = tbl.shape
    lpr = idx_rows.shape[0]
    per_tile = lpr // 32
    phases = per_tile // _CP
    assert per_tile == phases * _CP

    @functools.partial(
        pl.kernel,
        out_type=jax.ShapeDtypeStruct((lpr * _LANES, d), jnp.float32),
        mesh=_sc_mesh(),
        compiler_params=pltpu.CompilerParams(use_tc_tiling_on_sc=False),
        scratch_types=[
            pltpu.VMEM((_CP, _LANES), jnp.int32),
            pltpu.VMEM((_LANES, d), jnp.float32),
            pltpu.VMEM((_LANES, d), jnp.float32),
            pltpu.SemaphoreType.DMA,
            pltpu.SemaphoreType.DMA,
        ],
    )
    def k(t_hbm, idx_hbm, out_hbm, idx_v, rows0, rows1, sem0, sem1):
        c = lax.axis_index("c")
        s = lax.axis_index("s")
        base = (s * 2 + c) * per_tile
        for ph in range(phases):
            pb = base + ph * _CP
            pltpu.sync_copy(idx_hbm.at[pl.ds(pb, _CP)], idx_v)
            pltpu.async_copy(t_hbm.at[idx_v.at[0]], rows0, sem0)

            def pair(m, cc, pb=pb):
                j0 = 2 * m
                pltpu.async_copy(t_hbm.at[idx_v.at[j0 + 1]], rows1, sem1)
                pltpu.make_async_copy(
                    t_hbm.at[idx_v.at[0]], rows0, sem0).wait()
                pltpu.sync_copy(
                    rows0, out_hbm.at[pl.ds((pb + j0) * _LANES, _LANES)])

                @pl.when(m < _CP // 2 - 1)
                def _():
                    pltpu.async_copy(t_hbm.at[idx_v.at[j0 + 2]], rows0, sem0)

                pltpu.make_async_copy(
                    t_hbm.at[idx_v.at[0]], rows1, sem1).wait()
                pltpu.sync_copy(
                    rows1, out_hbm.at[pl.ds((pb + j0 + 1) * _LANES, _LANES)])
                return cc

            lax.fori_loop(0, _CP // 2, pair, 0)

    return k(tbl, idx_rows)


def _tc_first_call(deg2, xp, w1):
    npad, f = xp.shape
    dn = w1.shape[1]
    grid = npad // _BN

    def body(d_ref, x_ref, w_ref, dinv_ref, g_ref):
        deg = d_ref[0] + d_ref[1] + 1.0
        dinv = lax.rsqrt(deg)
        h = jnp.dot(x_ref[...], w_ref[...], preferred_element_type=jnp.float32)
        dinv_ref[...] = dinv[:, None]
        g_ref[...] = h * dinv[:, None]

    return pl.pallas_call(
        body,
        grid=(grid,),
        in_specs=[pl.BlockSpec((2, _BN), lambda i: (0, i)),
                  pl.BlockSpec((_BN, f), lambda i: (i, 0)),
                  pl.BlockSpec(w1.shape, lambda i: (0, 0))],
        out_specs=[pl.BlockSpec((_BN, 1), lambda i: (i, 0)),
                   pl.BlockSpec((_BN, dn), lambda i: (i, 0))],
        out_shape=[jax.ShapeDtypeStruct((npad, 1), jnp.float32),
                   jax.ShapeDtypeStruct((npad, dn), jnp.float32)],
    )(deg2, xp, w1)


def _tc_layer_call(acc, gprev, dinv, b, w, split):
    _, npad, di = acc.shape
    dn = w.shape[1]
    grid = npad // _BN

    def body(a_ref, g_ref, di_ref, b_ref, w_ref, *outs):
        dv = di_ref[...]
        h = jax.nn.relu(dv * (a_ref[0] + a_ref[1] + g_ref[...])
                        + b_ref[...])
        gn = jnp.dot(h, w_ref[...],
                     preferred_element_type=jnp.float32) * dv
        if split:
            outs[0][...] = gn[:, :dn // 2]
            outs[1][...] = gn[:, dn // 2:]
        else:
            outs[0][...] = gn

    if split:
        out_specs = [pl.BlockSpec((_BN, dn // 2), lambda i: (i, 0)),
                     pl.BlockSpec((_BN, dn // 2), lambda i: (i, 0))]
        out_shape = [jax.ShapeDtypeStruct((npad, dn // 2), jnp.float32),
                     jax.ShapeDtypeStruct((npad, dn // 2), jnp.float32)]
    else:
        out_specs = [pl.BlockSpec((_BN, dn), lambda i: (i, 0))]
        out_shape = [jax.ShapeDtypeStruct((npad, dn), jnp.float32)]

    return pl.pallas_call(
        body,
        grid=(grid,),
        in_specs=[pl.BlockSpec((2, _BN, di), lambda i: (0, i, 0)),
                  pl.BlockSpec((_BN, di), lambda i: (i, 0)),
                  pl.BlockSpec((_BN, 1), lambda i: (i, 0)),
                  pl.BlockSpec(b.shape, lambda i: (0,)),
                  pl.BlockSpec(w.shape, lambda i: (0, 0))],
        out_specs=out_specs,
        out_shape=out_shape,
    )(acc, gprev, dinv, b, w)


def _tc_pq_call(acc3, g3a, g3b, dinv, b3, lw1):
    _, npad, dh = acc3.shape
    d4 = lw1.shape[1]
    grid = npad // _BN

    def body(a_ref, ga_ref, gb_ref, di_ref, b_ref, w_ref, o_ref):
        dv = di_ref[...]
        bb = b_ref[...]
        w = w_ref[...]
        ha = jax.nn.relu(dv * (a_ref[0] + ga_ref[...]) + bb[:dh])
        hb = jax.nn.relu(dv * (a_ref[1] + gb_ref[...]) + bb[dh:])
        p = (jnp.dot(ha, w[:dh], preferred_element_type=jnp.float32)
             + jnp.dot(hb, w[dh:2 * dh], preferred_element_type=jnp.float32))
        q = (jnp.dot(ha, w[2 * dh:3 * dh], preferred_element_type=jnp.float32)
             + jnp.dot(hb, w[3 * dh:], preferred_element_type=jnp.float32))
        o_ref[0] = p
        o_ref[1] = q

    return pl.pallas_call(
        body,
        grid=(grid,),
        in_specs=[pl.BlockSpec((2, _BN, dh), lambda i: (0, i, 0)),
                  pl.BlockSpec((_BN, dh), lambda i: (i, 0)),
                  pl.BlockSpec((_BN, dh), lambda i: (i, 0)),
                  pl.BlockSpec((_BN, 1), lambda i: (i, 0)),
                  pl.BlockSpec((2 * dh,), lambda i: (0,)),
                  pl.BlockSpec(lw1.shape, lambda i: (0, 0))],
        out_specs=pl.BlockSpec((2, _BN, d4), lambda i: (0, i, 0)),
        out_shape=jax.ShapeDtypeStruct((2, npad, d4), jnp.float32),
    )(acc3, g3a, g3b, dinv, b3, lw1)


def _tc_link_call(gath, lb1, lw2, lb2, ew):
    d4 = gath.shape[1]
    do = lw2.shape[1]
    grid = ew // _BE

    def body(a_ref, b_ref, l1, w2, l2, o_ref):
        hidden = a_ref[...] + b_ref[...] + l1[...]
        o_ref[...] = jnp.dot(hidden, w2[...],
                             preferred_element_type=jnp.float32) + l2[...]

    return pl.pallas_call(
        body,
        grid=(grid,),
        in_specs=[pl.BlockSpec((_BE, d4), lambda i: (i, 0)),
                  pl.BlockSpec((_BE, d4), lambda i, g=grid: (i + g, 0)),
                  pl.BlockSpec((d4,), lambda i: (0,)),
                  pl.BlockSpec((d4, do), lambda i: (0, 0)),
                  pl.BlockSpec((do,), lambda i: (0,))],
        out_specs=pl.BlockSpec((_BE, do), lambda i: (i, 0)),
        out_shape=jax.ShapeDtypeStruct((ew, do), jnp.float32),
    )(gath, gath, lb1, lw2, lb2)


def kernel(x, edge_index, w2b, W1, b1, W2, b2, W3, b3, LW1, LB1, LW2, LB2):
    n, _ = x.shape
    e = edge_index.shape[1]
    ew = w2b.shape[1]

    npad = ((n + 1 + _BN - 1) // _BN) * _BN
    eq = 2 * 16 * _CP * _LANES
    epad = ((e + eq - 1) // eq) * eq
    lpad = ((2 * ew + eq - 1) // eq) * eq

    idx_pad = jnp.full((epad - e,), n, jnp.int32)
    srcp = jnp.concatenate([edge_index[0], idx_pad]).reshape(-1, _LANES)
    dstp = jnp.concatenate([edge_index[1], idx_pad]).reshape(-1, _LANES)
    lidx = jnp.concatenate(
        [w2b[0], w2b[1] + npad,
         jnp.full((lpad - 2 * ew,), n, jnp.int32)]).reshape(-1, _LANES)
    xp = jnp.pad(x, ((0, npad - n), (0, 0)))

    z1 = jnp.zeros((npad,), jnp.float32)
    zA = jnp.zeros((npad, W1.shape[1]), jnp.float32)
    zB = jnp.zeros((npad, W2.shape[1]), jnp.float32)

    deg2 = _deg_call(dstp, z1).reshape(2, npad)
    dinv, g1 = _tc_first_call(deg2, xp, W1)
    acc1 = _agg_edge_call(g1, srcp, dstp, zA)
    (g2,) = _tc_layer_call(acc1, g1, dinv, b1, W2, split=False)
    acc2 = _agg_edge_call(g2, srcp, dstp, zB)
    g3a, g3b = _tc_layer_call(acc2, g2, dinv, b2, W3, split=True)
    acc3 = _agg_col_call(g3a, g3b, srcp, dstp, zB)
    dp = 16 - LW1.shape[1]
    lw1p = jnp.pad(LW1, ((0, 0), (0, dp)))
    lb1p = jnp.pad(LB1, (0, dp))
    lw2p = jnp.pad(LW2, ((0, dp), (0, 0)))
    pq = _tc_pq_call(acc3, g3a, g3b, dinv, b3, lw1p)
    gath = _link_gather_call(pq.reshape(2 * npad, -1), lidx)
    return _tc_link_call(gath, lb1p, lw2p, LB2, ew)


# speedup vs baseline: 19.7566x; 1.1134x over previous
import functools

import jax
import jax.numpy as jnp
from jax import lax
from jax.experimental import pallas as pl
from jax.experimental.pallas import tpu as pltpu
from jax.experimental.pallas import tpu_sc as plsc

_LANES = 128
_CP = 56
_BN = 3584
_BE = 6400


def _sc_mesh():
    return plsc.VectorSubcoreMesh(core_axis_name="c", subcore_axis_name="s")


def _deg_call(dst_rows, zeros1):
    npad = zeros1.shape[0]
    epr = dst_rows.shape[0]
    per_sc = epr // 2
    per_tile = per_sc // 16
    phases = per_tile // _CP
    assert per_tile == phases * _CP
    rows_out = npad // 16

    @functools.partial(
        pl.kernel,
        out_type=jax.ShapeDtypeStruct((2 * npad,), jnp.float32),
        mesh=_sc_mesh(),
        compiler_params=pltpu.CompilerParams(use_tc_tiling_on_sc=False),
        scratch_types=[
            pltpu.VMEM((_CP, _LANES), jnp.int32),
            pltpu.VMEM((_LANES,), jnp.float32),
            pltpu.VMEM_SHARED((npad,), jnp.float32),
        ],
    )
    def k(dst_hbm, z_hbm, out_hbm, idx_v, ones_v, acc):
        c = lax.axis_index("c")
        s = lax.axis_index("s")
        for i in range(_LANES // 16):
            ones_v[pl.ds(16 * i, 16)] = jnp.ones((16,), jnp.float32)
        pltpu.sync_copy(z_hbm.at[pl.ds(s * rows_out, rows_out)],
                        acc.at[pl.ds(s * rows_out, rows_out)])
        plsc.subcore_barrier()
        base = c * per_sc + s * per_tile
        for ph in range(phases):
            pltpu.sync_copy(dst_hbm.at[pl.ds(base + ph * _CP, _CP)], idx_v)

            def body(j, cc):
                pltpu.sync_copy(ones_v, acc.at[idx_v.at[j]], add=True)
                return cc

            lax.fori_loop(0, _CP, body, 0)
        plsc.subcore_barrier()
        pltpu.sync_copy(acc.at[pl.ds(s * rows_out, rows_out)],
                        out_hbm.at[pl.ds(c * npad + s * rows_out, rows_out)])

    return k(dst_rows, zeros1)


def _agg_pipeline(g_hbm, src_hbm, dst_hbm, acc, idxs_v, idxd_v,
                  bufs, gsems, ssems, base, phases):
    nb = len(bufs)
    for ph in range(phases):
        pb = base + ph * _CP
        pltpu.sync_copy(src_hbm.at[pl.ds(pb, _CP)], idxs_v)
        pltpu.sync_copy(dst_hbm.at[pl.ds(pb, _CP)], idxd_v)
        for t in range(nb):
            pltpu.async_copy(g_hbm.at[idxs_v.at[t]], bufs[t], gsems[t])

        def group(m, cc):
            j0 = nb * m
            for t in range(nb):
                pltpu.make_async_copy(
                    g_hbm.at[idxs_v.at[0]], bufs[t], gsems[t]).wait()
                pltpu.async_copy(
                    bufs[t], acc.at[idxd_v.at[j0 + t]], ssems[t], add=True)

            @pl.when(m < _CP // nb - 1)
            def _():
                for t in range(nb):
                    pltpu.make_async_copy(
                        bufs[t], acc.at[idxd_v.at[0]], ssems[t]).wait()
                    pltpu.async_copy(
                        g_hbm.at[idxs_v.at[j0 + nb + t]], bufs[t], gsems[t])
            return cc

        lax.fori_loop(0, _CP // nb, group, 0)
        for t in range(nb):
            pltpu.make_async_copy(
                bufs[t], acc.at[idxd_v.at[0]], ssems[t]).wait()


def _agg_edge_call(g, src_rows, dst_rows, zeros_d):
    npad, d = g.shape
    epr = src_rows.shape[0]
    per_sc = epr // 2
    per_tile = per_sc // 16
    phases = per_tile // _CP
    assert per_tile == phases * _CP
    rows_out = npad // 16

    @functools.partial(
        pl.kernel,
        out_type=jax.ShapeDtypeStruct((2, npad, d), jnp.float32),
        mesh=_sc_mesh(),
        compiler_params=pltpu.CompilerParams(use_tc_tiling_on_sc=False),
        scratch_types=[
            pltpu.VMEM((_CP, _LANES), jnp.int32),
            pltpu.VMEM((_CP, _LANES), jnp.int32),
            [pltpu.VMEM((_LANES, d), jnp.float32)] * 4,
            pltpu.VMEM_SHARED((npad, d), jnp.float32),
            [pltpu.SemaphoreType.DMA] * 4,
            [pltpu.SemaphoreType.DMA] * 4,
        ],
    )
    def k(g_hbm, src_hbm, dst_hbm, z_hbm, out_hbm,
          idxs_v, idxd_v, bufs, acc, gsems, ssems):
        c = lax.axis_index("c")
        s = lax.axis_index("s")
        pltpu.sync_copy(z_hbm.at[pl.ds(s * rows_out, rows_out)],
                        acc.at[pl.ds(s * rows_out, rows_out)])
        plsc.subcore_barrier()
        base = c * per_sc + s * per_tile
        _agg_pipeline(g_hbm, src_hbm, dst_hbm, acc, idxs_v, idxd_v,
                      bufs, gsems, ssems, base, phases=phases)
        plsc.subcore_barrier()
        pltpu.sync_copy(acc.at[pl.ds(s * rows_out, rows_out)],
                        out_hbm.at[c, pl.ds(s * rows_out, rows_out)])

    return k(g, src_rows, dst_rows, zeros_d)


def _agg_col_call(ga, gb, src_rows, dst_rows, zeros_d):
    npad, d = ga.shape
    epr = src_rows.shape[0]
    per_tile = epr // 16
    phases = per_tile // _CP
    assert per_tile == phases * _CP
    rows_out = npad // 16

    @functools.partial(
        pl.kernel,
        out_type=jax.ShapeDtypeStruct((2, npad, d), jnp.float32),
        mesh=_sc_mesh(),
        compiler_params=pltpu.CompilerParams(use_tc_tiling_on_sc=False),
        scratch_types=[
            pltpu.VMEM((_CP, _LANES), jnp.int32),
            pltpu.VMEM((_CP, _LANES), jnp.int32),
            [pltpu.VMEM((_LANES, d), jnp.float32)] * 4,
            pltpu.VMEM_SHARED((npad, d), jnp.float32),
            [pltpu.SemaphoreType.DMA] * 4,
            [pltpu.SemaphoreType.DMA] * 4,
        ],
    )
    def k(ga_hbm, gb_hbm, src_hbm, dst_hbm, z_hbm, out_hbm,
          idxs_v, idxd_v, bufs, acc, gsems, ssems):
        c = lax.axis_index("c")
        s = lax.axis_index("s")
        pltpu.sync_copy(z_hbm.at[pl.ds(s * rows_out, rows_out)],
                        acc.at[pl.ds(s * rows_out, rows_out)])
        plsc.subcore_barrier()
        base = s * per_tile

        @pl.when(c == 0)
        def _():
            _agg_pipeline(ga_hbm, src_hbm, dst_hbm, acc, idxs_v, idxd_v,
                          bufs, gsems, ssems, base, phases=phases)

        @pl.when(c != 0)
        def _():
            _agg_pipeline(gb_hbm, src_hbm, dst_hbm, acc, idxs_v, idxd_v,
                          bufs, gsems, ssems, base, phases=phases)

        plsc.subcore_barrier()
        pltpu.sync_copy(acc.at[pl.ds(s * rows_out, rows_out)],
                        out_hbm.at[c, pl.ds(s * rows_out, rows_out)])

    return k(ga, gb, src_rows, dst_rows, zeros_d)


def _link_gather_call(tbl, idx_rows):
    _, d = tbl.shape
    lpr = idx_rows.shape[0]
    per_tile = lpr // 32
    phases = per_tile // _CP
    assert per_tile == phases * _CP

    @functools.partial(
        pl.kernel,
        out_type=jax.ShapeDtypeStruct((lpr * _LANES, d), jnp.float32),
        mesh=_sc_mesh(),
        compiler_params=pltpu.CompilerParams(use_tc_tiling_on_sc=False),
        scratch_types=[
            pltpu.VMEM((_CP, _LANES), jnp.int32),
            [pltpu.VMEM((_LANES, d), jnp.float32)] * 4,
            [pltpu.SemaphoreType.DMA] * 4,
            [pltpu.SemaphoreType.DMA] * 4,
        ],
    )
    def k(t_hbm, idx_hbm, out_hbm, idx_v, bufs, gsems, ssems):
        c = lax.axis_index("c")
        s = lax.axis_index("s")
        base = (s * 2 + c) * per_tile
        nb = len(bufs)
        for ph in range(phases):
            pb = base + ph * _CP
            pltpu.sync_copy(idx_hbm.at[pl.ds(pb, _CP)], idx_v)
            for t in range(nb):
                pltpu.async_copy(t_hbm.at[idx_v.at[t]], bufs[t], gsems[t])

            def group(m, cc, pb=pb):
                j0 = nb * m
                for t in range(nb):
                    pltpu.make_async_copy(
                        t_hbm.at[idx_v.at[0]], bufs[t], gsems[t]).wait()
                    pltpu.async_copy(
                        bufs[t],
                        out_hbm.at[pl.ds((pb + j0 + t) * _LANES, _LANES)],
                        ssems[t])

                @pl.when(m < _CP // nb - 1)
                def _():
                    for t in range(nb):
                        pltpu.make_async_copy(
                            bufs[t], out_hbm.at[pl.ds(0, _LANES)],
                            ssems[t]).wait()
                        pltpu.async_copy(
                            t_hbm.at[idx_v.at[j0 + nb + t]], bufs[t],
                            gsems[t])
                return cc

            lax.fori_loop(0, _CP // nb, group, 0)
            for t in range(nb):
                pltpu.make_async_copy(
                    bufs[t], out_hbm.at[pl.ds(0, _LANES)], ssems[t]).wait()

    return k(tbl, idx_rows)


def _tc_first_call(deg2, xp, w1):
    npad, f = xp.shape
    dn = w1.shape[1]
    grid = npad // _BN

    def body(d_ref, x_ref, w_ref, dinv_ref, g_ref):
        deg = d_ref[0] + d_ref[1] + 1.0
        dinv = lax.rsqrt(deg)
        h = jnp.dot(x_ref[...], w_ref[...], preferred_element_type=jnp.float32)
        dinv_ref[...] = dinv[:, None]
        g_ref[...] = h * dinv[:, None]

    return pl.pallas_call(
        body,
        grid=(grid,),
        in_specs=[pl.BlockSpec((2, _BN), lambda i: (0, i)),
                  pl.BlockSpec((_BN, f), lambda i: (i, 0)),
                  pl.BlockSpec(w1.shape, lambda i: (0, 0))],
        out_specs=[pl.BlockSpec((_BN, 1), lambda i: (i, 0)),
                   pl.BlockSpec((_BN, dn), lambda i: (i, 0))],
        out_shape=[jax.ShapeDtypeStruct((npad, 1), jnp.float32),
                   jax.ShapeDtypeStruct((npad, dn), jnp.float32)],
    )(deg2, xp, w1)


def _tc_layer_call(acc, gprev, dinv, b, w, split):
    _, npad, di = acc.shape
    dn = w.shape[1]
    grid = npad // _BN

    def body(a_ref, g_ref, di_ref, b_ref, w_ref, *outs):
        dv = di_ref[...]
        h = jax.nn.relu(dv * (a_ref[0] + a_ref[1] + g_ref[...])
                        + b_ref[...])
        gn = jnp.dot(h, w_ref[...],
                     preferred_element_type=jnp.float32) * dv
        if split:
            outs[0][...] = gn[:, :dn // 2]
            outs[1][...] = gn[:, dn // 2:]
        else:
            outs[0][...] = gn

    if split:
        out_specs = [pl.BlockSpec((_BN, dn // 2), lambda i: (i, 0)),
                     pl.BlockSpec((_BN, dn // 2), lambda i: (i, 0))]
        out_shape = [jax.ShapeDtypeStruct((npad, dn // 2), jnp.float32),
                     jax.ShapeDtypeStruct((npad, dn // 2), jnp.float32)]
    else:
        out_specs = [pl.BlockSpec((_BN, dn), lambda i: (i, 0))]
        out_shape = [jax.ShapeDtypeStruct((npad, dn), jnp.float32)]

    return pl.pallas_call(
        body,
        grid=(grid,),
        in_specs=[pl.BlockSpec((2, _BN, di), lambda i: (0, i, 0)),
                  pl.BlockSpec((_BN, di), lambda i: (i, 0)),
                  pl.BlockSpec((_BN, 1), lambda i: (i, 0)),
                  pl.BlockSpec(b.shape, lambda i: (0,)),
                  pl.BlockSpec(w.shape, lambda i: (0, 0))],
        out_specs=out_specs,
        out_shape=out_shape,
    )(acc, gprev, dinv, b, w)


def _tc_pq_call(acc3, g3a, g3b, dinv, b3, lw1):
    _, npad, dh = acc3.shape
    d4 = lw1.shape[1]
    grid = npad // _BN

    def body(a_ref, ga_ref, gb_ref, di_ref, b_ref, w_ref, o_ref):
        dv = di_ref[...]
        bb = b_ref[...]
        w = w_ref[...]
        ha = jax.nn.relu(dv * (a_ref[0] + ga_ref[...]) + bb[:dh])
        hb = jax.nn.relu(dv * (a_ref[1] + gb_ref[...]) + bb[dh:])
        p = (jnp.dot(ha, w[:dh], preferred_element_type=jnp.float32)
             + jnp.dot(hb, w[dh:2 * dh], preferred_element_type=jnp.float32))
        q = (jnp.dot(ha, w[2 * dh:3 * dh], preferred_element_type=jnp.float32)
             + jnp.dot(hb, w[3 * dh:], preferred_element_type=jnp.float32))
        o_ref[0] = p
        o_ref[1] = q

    return pl.pallas_call(
        body,
        grid=(grid,),
        in_specs=[pl.BlockSpec((2, _BN, dh), lambda i: (0, i, 0)),
                  pl.BlockSpec((_BN, dh), lambda i: (i, 0)),
                  pl.BlockSpec((_BN, dh), lambda i: (i, 0)),
                  pl.BlockSpec((_BN, 1), lambda i: (i, 0)),
                  pl.BlockSpec((2 * dh,), lambda i: (0,)),
                  pl.BlockSpec(lw1.shape, lambda i: (0, 0))],
        out_specs=pl.BlockSpec((2, _BN, d4), lambda i: (0, i, 0)),
        out_shape=jax.ShapeDtypeStruct((2, npad, d4), jnp.float32),
    )(acc3, g3a, g3b, dinv, b3, lw1)


def _tc_link_call(gath, lb1, lw2, lb2, ew):
    d4 = gath.shape[1]
    do = lw2.shape[1]
    grid = ew // _BE

    def body(a_ref, b_ref, l1, w2, l2, o_ref):
        hidden = a_ref[...] + b_ref[...] + l1[...]
        o_ref[...] = jnp.dot(hidden, w2[...],
                             preferred_element_type=jnp.float32) + l2[...]

    return pl.pallas_call(
        body,
        grid=(grid,),
        in_specs=[pl.BlockSpec((_BE, d4), lambda i: (i, 0)),
                  pl.BlockSpec((_BE, d4), lambda i, g=grid: (i + g, 0)),
                  pl.BlockSpec((d4,), lambda i: (0,)),
                  pl.BlockSpec((d4, do), lambda i: (0, 0)),
                  pl.BlockSpec((do,), lambda i: (0,))],
        out_specs=pl.BlockSpec((_BE, do), lambda i: (i, 0)),
        out_shape=jax.ShapeDtypeStruct((ew, do), jnp.float32),
    )(gath, gath, lb1, lw2, lb2)


def kernel(x, edge_index, w2b, W1, b1, W2, b2, W3, b3, LW1, LB1, LW2, LB2):
    n, _ = x.shape
    e = edge_index.shape[1]
    ew = w2b.shape[1]

    npad = ((n + 1 + _BN - 1) // _BN) * _BN
    eq = 2 * 16 * _CP * _LANES
    epad = ((e + eq - 1) // eq) * eq
    lpad = ((2 * ew + eq - 1) // eq) * eq

    idx_pad = jnp.full((epad - e,), n, jnp.int32)
    srcp = jnp.concatenate([edge_index[0], idx_pad]).reshape(-1, _LANES)
    dstp = jnp.concatenate([edge_index[1], idx_pad]).reshape(-1, _LANES)
    lidx = jnp.concatenate(
        [w2b[0], w2b[1] + npad,
         jnp.full((lpad - 2 * ew,), n, jnp.int32)]).reshape(-1, _LANES)
    xp = jnp.pad(x, ((0, npad - n), (0, 0)))

    z1 = jnp.zeros((npad,), jnp.float32)
    zA = jnp.zeros((npad, W1.shape[1]), jnp.float32)
    zB = jnp.zeros((npad, W2.shape[1]), jnp.float32)

    deg2 = _deg_call(dstp, z1).reshape(2, npad)
    dinv, g1 = _tc_first_call(deg2, xp, W1)
    acc1 = _agg_edge_call(g1, srcp, dstp, zA)
    (g2,) = _tc_layer_call(acc1, g1, dinv, b1, W2, split=False)
    acc2 = _agg_edge_call(g2, srcp, dstp, zB)
    g3a, g3b = _tc_layer_call(acc2, g2, dinv, b2, W3, split=True)
    acc3 = _agg_col_call(g3a, g3b, srcp, dstp, zB)
    dp = 16 - LW1.shape[1]
    lw1p = jnp.pad(LW1, ((0, 0), (0, dp)))
    lb1p = jnp.pad(LB1, (0, dp))
    lw2p = jnp.pad(LW2, ((0, dp), (0, 0)))
    pq = _tc_pq_call(acc3, g3a, g3b, dinv, b3, lw1p)
    gath = _link_gather_call(pq.reshape(2 * npad, -1), lidx)
    return _tc_link_call(gath, lb1p, lw2p, LB2, ew)


# speedup vs baseline: 19.7999x; 1.0022x over previous
import functools

import jax
import jax.numpy as jnp
from jax import lax
from jax.experimental import pallas as pl
from jax.experimental.pallas import tpu as pltpu
from jax.experimental.pallas import tpu_sc as plsc

_LANES = 128
_CP = 56
_BN = 3584
_BE = 6400


def _sc_mesh():
    return plsc.VectorSubcoreMesh(core_axis_name="c", subcore_axis_name="s")


def _deg_call(dst_rows, zeros1):
    npad = zeros1.shape[0]
    epr = dst_rows.shape[0]
    per_sc = epr // 2
    per_tile = per_sc // 16
    rows_out = npad // 16

    @functools.partial(
        pl.kernel,
        out_type=jax.ShapeDtypeStruct((2 * npad,), jnp.float32),
        mesh=_sc_mesh(),
        compiler_params=pltpu.CompilerParams(use_tc_tiling_on_sc=False),
        scratch_types=[
            pltpu.VMEM((per_tile, _LANES), jnp.int32),
            pltpu.VMEM((_LANES,), jnp.float32),
            pltpu.VMEM_SHARED((npad,), jnp.float32),
            pltpu.SemaphoreType.DMA,
        ],
    )
    def k(dst_hbm, z_hbm, out_hbm, idx_v, ones_v, acc, sem):
        c = lax.axis_index("c")
        s = lax.axis_index("s")
        for i in range(_LANES // 16):
            ones_v[pl.ds(16 * i, 16)] = jnp.ones((16,), jnp.float32)
        pltpu.sync_copy(z_hbm.at[pl.ds(s * rows_out, rows_out)],
                        acc.at[pl.ds(s * rows_out, rows_out)])
        plsc.subcore_barrier()
        base = c * per_sc + s * per_tile
        pltpu.sync_copy(dst_hbm.at[pl.ds(base, per_tile)], idx_v)

        def group(m, cc):
            for t in range(8):
                pltpu.async_copy(
                    ones_v, acc.at[idx_v.at[8 * m + t]], sem, add=True)
            for t in range(8):
                pltpu.make_async_copy(
                    ones_v, acc.at[idx_v.at[0]], sem).wait()
            return cc

        lax.fori_loop(0, per_tile // 8, group, 0)
        plsc.subcore_barrier()
        pltpu.sync_copy(acc.at[pl.ds(s * rows_out, rows_out)],
                        out_hbm.at[pl.ds(c * npad + s * rows_out, rows_out)])

    return k(dst_rows, zeros1)


def _agg_pipeline(g_hbm, src_hbm, dst_hbm, acc, idxs_v, idxd_v,
                  bufs, gsems, ssems, base, phases):
    nb = len(bufs)
    for ph in range(phases):
        pb = base + ph * _CP
        pltpu.sync_copy(src_hbm.at[pl.ds(pb, _CP)], idxs_v)
        pltpu.sync_copy(dst_hbm.at[pl.ds(pb, _CP)], idxd_v)
        for t in range(nb):
            pltpu.async_copy(g_hbm.at[idxs_v.at[t]], bufs[t], gsems[t])

        def group(m, cc):
            j0 = nb * m
            for t in range(nb):
                pltpu.make_async_copy(
                    g_hbm.at[idxs_v.at[0]], bufs[t], gsems[t]).wait()
                pltpu.async_copy(
                    bufs[t], acc.at[idxd_v.at[j0 + t]], ssems[t], add=True)

            @pl.when(m < _CP // nb - 1)
            def _():
                for t in range(nb):
                    pltpu.make_async_copy(
                        bufs[t], acc.at[idxd_v.at[0]], ssems[t]).wait()
                    pltpu.async_copy(
                        g_hbm.at[idxs_v.at[j0 + nb + t]], bufs[t], gsems[t])
            return cc

        lax.fori_loop(0, _CP // nb, group, 0)
        for t in range(nb):
            pltpu.make_async_copy(
                bufs[t], acc.at[idxd_v.at[0]], ssems[t]).wait()


def _agg_edge_call(g, src_rows, dst_rows, zeros_d):
    npad, d = g.shape
    epr = src_rows.shape[0]
    per_sc = epr // 2
    per_tile = per_sc // 16
    phases = per_tile // _CP
    assert per_tile == phases * _CP
    rows_out = npad // 16

    @functools.partial(
        pl.kernel,
        out_type=jax.ShapeDtypeStruct((2, npad, d), jnp.float32),
        mesh=_sc_mesh(),
        compiler_params=pltpu.CompilerParams(use_tc_tiling_on_sc=False),
        scratch_types=[
            pltpu.VMEM((_CP, _LANES), jnp.int32),
            pltpu.VMEM((_CP, _LANES), jnp.int32),
            [pltpu.VMEM((_LANES, d), jnp.float32)] * 4,
            pltpu.VMEM_SHARED((npad, d), jnp.float32),
            [pltpu.SemaphoreType.DMA] * 4,
            [pltpu.SemaphoreType.DMA] * 4,
        ],
    )
    def k(g_hbm, src_hbm, dst_hbm, z_hbm, out_hbm,
          idxs_v, idxd_v, bufs, acc, gsems, ssems):
        c = lax.axis_index("c")
        s = lax.axis_index("s")
        pltpu.sync_copy(z_hbm.at[pl.ds(s * rows_out, rows_out)],
                        acc.at[pl.ds(s * rows_out, rows_out)])
        plsc.subcore_barrier()
        base = c * per_sc + s * per_tile
        _agg_pipeline(g_hbm, src_hbm, dst_hbm, acc, idxs_v, idxd_v,
                      bufs, gsems, ssems, base, phases=phases)
        plsc.subcore_barrier()
        pltpu.sync_copy(acc.at[pl.ds(s * rows_out, rows_out)],
                        out_hbm.at[c, pl.ds(s * rows_out, rows_out)])

    return k(g, src_rows, dst_rows, zeros_d)


def _agg_col_call(ga, gb, src_rows, dst_rows, zeros_d):
    npad, d = ga.shape
    epr = src_rows.shape[0]
    per_tile = epr // 16
    phases = per_tile // _CP
    assert per_tile == phases * _CP
    rows_out = npad // 16

    @functools.partial(
        pl.kernel,
        out_type=jax.ShapeDtypeStruct((2, npad, d), jnp.float32),
        mesh=_sc_mesh(),
        compiler_params=pltpu.CompilerParams(use_tc_tiling_on_sc=False),
        scratch_types=[
            pltpu.VMEM((_CP, _LANES), jnp.int32),
            pltpu.VMEM((_CP, _LANES), jnp.int32),
            [pltpu.VMEM((_LANES, d), jnp.float32)] * 4,
            pltpu.VMEM_SHARED((npad, d), jnp.float32),
            [pltpu.SemaphoreType.DMA] * 4,
            [pltpu.SemaphoreType.DMA] * 4,
        ],
    )
    def k(ga_hbm, gb_hbm, src_hbm, dst_hbm, z_hbm, out_hbm,
          idxs_v, idxd_v, bufs, acc, gsems, ssems):
        c = lax.axis_index("c")
        s = lax.axis_index("s")
        pltpu.sync_copy(z_hbm.at[pl.ds(s * rows_out, rows_out)],
                        acc.at[pl.ds(s * rows_out, rows_out)])
        plsc.subcore_barrier()
        base = s * per_tile

        @pl.when(c == 0)
        def _():
            _agg_pipeline(ga_hbm, src_hbm, dst_hbm, acc, idxs_v, idxd_v,
                          bufs, gsems, ssems, base, phases=phases)

        @pl.when(c != 0)
        def _():
            _agg_pipeline(gb_hbm, src_hbm, dst_hbm, acc, idxs_v, idxd_v,
                          bufs, gsems, ssems, base, phases=phases)

        plsc.subcore_barrier()
        pltpu.sync_copy(acc.at[pl.ds(s * rows_out, rows_out)],
                        out_hbm.at[c, pl.ds(s * rows_out, rows_out)])

    return k(ga, gb, src_rows, dst_rows, zeros_d)


def _link_gather_call(tbl, idx_rows):
    _, d = tbl.shape
    lpr = idx_rows.shape[0]
    per_tile = lpr // 32

    @functools.partial(
        pl.kernel,
        out_type=jax.ShapeDtypeStruct((lpr * _LANES, d), jnp.float32),
        mesh=_sc_mesh(),
        compiler_params=pltpu.CompilerParams(use_tc_tiling_on_sc=False),
        scratch_types=[
            pltpu.VMEM((per_tile, _LANES), jnp.int32),
            [pltpu.VMEM((_LANES, d), jnp.float32)] * 4,
            [pltpu.SemaphoreType.DMA] * 4,
            [pltpu.SemaphoreType.DMA] * 4,
        ],
    )
    def k(t_hbm, idx_hbm, out_hbm, idx_v, bufs, gsems, ssems):
        c = lax.axis_index("c")
        s = lax.axis_index("s")
        base = (s * 2 + c) * per_tile
        nb = len(bufs)
        pltpu.sync_copy(idx_hbm.at[pl.ds(base, per_tile)], idx_v)
        for t in range(nb):
            pltpu.async_copy(t_hbm.at[idx_v.at[t]], bufs[t], gsems[t])

        def group(m, cc):
            j0 = nb * m
            for t in range(nb):
                pltpu.make_async_copy(
                    t_hbm.at[idx_v.at[0]], bufs[t], gsems[t]).wait()
                pltpu.async_copy(
                    bufs[t],
                    out_hbm.at[pl.ds((base + j0 + t) * _LANES, _LANES)],
                    ssems[t])

            @pl.when(m < per_tile // nb - 1)
            def _():
                for t in range(nb):
                    pltpu.make_async_copy(
                        bufs[t], out_hbm.at[pl.ds(0, _LANES)],
                        ssems[t]).wait()
                    pltpu.async_copy(
                        t_hbm.at[idx_v.at[j0 + nb + t]], bufs[t],
                        gsems[t])
            return cc

        lax.fori_loop(0, per_tile // nb, group, 0)
        for t in range(nb):
            pltpu.make_async_copy(
                bufs[t], out_hbm.at[pl.ds(0, _LANES)], ssems[t]).wait()

    return k(tbl, idx_rows)


def _tc_first_call(deg2, xp, w1):
    npad, f = xp.shape
    dn = w1.shape[1]
    grid = npad // _BN

    def body(d_ref, x_ref, w_ref, dinv_ref, g_ref):
        deg = d_ref[0] + d_ref[1] + 1.0
        dinv = lax.rsqrt(deg)
        h = jnp.dot(x_ref[...], w_ref[...], preferred_element_type=jnp.float32)
        dinv_ref[...] = dinv[:, None]
        g_ref[...] = h * dinv[:, None]

    return pl.pallas_call(
        body,
        grid=(grid,),
        in_specs=[pl.BlockSpec((2, _BN), lambda i: (0, i)),
                  pl.BlockSpec((_BN, f), lambda i: (i, 0)),
                  pl.BlockSpec(w1.shape, lambda i: (0, 0))],
        out_specs=[pl.BlockSpec((_BN, 1), lambda i: (i, 0)),
                   pl.BlockSpec((_BN, dn), lambda i: (i, 0))],
        out_shape=[jax.ShapeDtypeStruct((npad, 1), jnp.float32),
                   jax.ShapeDtypeStruct((npad, dn), jnp.float32)],
    )(deg2, xp, w1)


def _tc_layer_call(acc, gprev, dinv, b, w, split):
    _, npad, di = acc.shape
    dn = w.shape[1]
    grid = npad // _BN

    def body(a_ref, g_ref, di_ref, b_ref, w_ref, *outs):
        dv = di_ref[...]
        h = jax.nn.relu(dv * (a_ref[0] + a_ref[1] + g_ref[...])
                        + b_ref[...])
        gn = jnp.dot(h, w_ref[...],
                     preferred_element_type=jnp.float32) * dv
        if split:
            outs[0][...] = gn[:, :dn // 2]
            outs[1][...] = gn[:, dn // 2:]
        else:
            outs[0][...] = gn

    if split:
        out_specs = [pl.BlockSpec((_BN, dn // 2), lambda i: (i, 0)),
                     pl.BlockSpec((_BN, dn // 2), lambda i: (i, 0))]
        out_shape = [jax.ShapeDtypeStruct((npad, dn // 2), jnp.float32),
                     jax.ShapeDtypeStruct((npad, dn // 2), jnp.float32)]
    else:
        out_specs = [pl.BlockSpec((_BN, dn), lambda i: (i, 0))]
        out_shape = [jax.ShapeDtypeStruct((npad, dn), jnp.float32)]

    return pl.pallas_call(
        body,
        grid=(grid,),
        in_specs=[pl.BlockSpec((2, _BN, di), lambda i: (0, i, 0)),
                  pl.BlockSpec((_BN, di), lambda i: (i, 0)),
                  pl.BlockSpec((_BN, 1), lambda i: (i, 0)),
                  pl.BlockSpec(b.shape, lambda i: (0,)),
                  pl.BlockSpec(w.shape, lambda i: (0, 0))],
        out_specs=out_specs,
        out_shape=out_shape,
    )(acc, gprev, dinv, b, w)


def _tc_pq_call(acc3, g3a, g3b, dinv, b3, lw1):
    _, npad, dh = acc3.shape
    d4 = lw1.shape[1]
    grid = npad // _BN

    def body(a_ref, ga_ref, gb_ref, di_ref, b_ref, w_ref, o_ref):
        dv = di_ref[...]
        bb = b_ref[...]
        w = w_ref[...]
        ha = jax.nn.relu(dv * (a_ref[0] + ga_ref[...]) + bb[:dh])
        hb = jax.nn.relu(dv * (a_ref[1] + gb_ref[...]) + bb[dh:])
        p = (jnp.dot(ha, w[:dh], preferred_element_type=jnp.float32)
             + jnp.dot(hb, w[dh:2 * dh], preferred_element_type=jnp.float32))
        q = (jnp.dot(ha, w[2 * dh:3 * dh], preferred_element_type=jnp.float32)
             + jnp.dot(hb, w[3 * dh:], preferred_element_type=jnp.float32))
        o_ref[0] = p
        o_ref[1] = q

    return pl.pallas_call(
        body,
        grid=(grid,),
        in_specs=[pl.BlockSpec((2, _BN, dh), lambda i: (0, i, 0)),
                  pl.BlockSpec((_BN, dh), lambda i: (i, 0)),
                  pl.BlockSpec((_BN, dh), lambda i: (i, 0)),
                  pl.BlockSpec((_BN, 1), lambda i: (i, 0)),
                  pl.BlockSpec((2 * dh,), lambda i: (0,)),
                  pl.BlockSpec(lw1.shape, lambda i: (0, 0))],
        out_specs=pl.BlockSpec((2, _BN, d4), lambda i: (0, i, 0)),
        out_shape=jax.ShapeDtypeStruct((2, npad, d4), jnp.float32),
    )(acc3, g3a, g3b, dinv, b3, lw1)


def _tc_link_call(gath, lb1, lw2, lb2, ew):
    d4 = gath.shape[1]
    do = lw2.shape[1]
    grid = ew // _BE

    def body(a_ref, b_ref, l1, w2, l2, o_ref):
        hidden = a_ref[...] + b_ref[...] + l1[...]
        o_ref[...] = jnp.dot(hidden, w2[...],
                             preferred_element_type=jnp.float32) + l2[...]

    return pl.pallas_call(
        body,
        grid=(grid,),
        in_specs=[pl.BlockSpec((_BE, d4), lambda i: (i, 0)),
                  pl.BlockSpec((_BE, d4), lambda i, g=grid: (i + g, 0)),
                  pl.BlockSpec((d4,), lambda i: (0,)),
                  pl.BlockSpec((d4, do), lambda i: (0, 0)),
                  pl.BlockSpec((do,), lambda i: (0,))],
        out_specs=pl.BlockSpec((_BE, do), lambda i: (i, 0)),
        out_shape=jax.ShapeDtypeStruct((ew, do), jnp.float32),
    )(gath, gath, lb1, lw2, lb2)


def kernel(x, edge_index, w2b, W1, b1, W2, b2, W3, b3, LW1, LB1, LW2, LB2):
    n, _ = x.shape
    e = edge_index.shape[1]
    ew = w2b.shape[1]

    npad = ((n + 1 + _BN - 1) // _BN) * _BN
    eq = 2 * 16 * _CP * _LANES
    epad = ((e + eq - 1) // eq) * eq
    lpad = ((2 * ew + eq - 1) // eq) * eq

    idx_pad = jnp.full((epad - e,), n, jnp.int32)
    srcp = jnp.concatenate([edge_index[0], idx_pad]).reshape(-1, _LANES)
    dstp = jnp.concatenate([edge_index[1], idx_pad]).reshape(-1, _LANES)
    lidx = jnp.concatenate(
        [w2b[0], w2b[1] + npad,
         jnp.full((lpad - 2 * ew,), n, jnp.int32)]).reshape(-1, _LANES)
    xp = jnp.pad(x, ((0, npad - n), (0, 0)))

    z1 = jnp.zeros((npad,), jnp.float32)
    zA = jnp.zeros((npad, W1.shape[1]), jnp.float32)
    zB = jnp.zeros((npad, W2.shape[1]), jnp.float32)

    deg2 = _deg_call(dstp, z1).reshape(2, npad)
    dinv, g1 = _tc_first_call(deg2, xp, W1)
    acc1 = _agg_edge_call(g1, srcp, dstp, zA)
    (g2,) = _tc_layer_call(acc1, g1, dinv, b1, W2, split=False)
    acc2 = _agg_edge_call(g2, srcp, dstp, zB)
    g3a, g3b = _tc_layer_call(acc2, g2, dinv, b2, W3, split=True)
    acc3 = _agg_col_call(g3a, g3b, srcp, dstp, zB)
    dp = 16 - LW1.shape[1]
    lw1p = jnp.pad(LW1, ((0, 0), (0, dp)))
    lb1p = jnp.pad(LB1, (0, dp))
    lw2p = jnp.pad(LW2, ((0, dp), (0, 0)))
    pq = _tc_pq_call(acc3, g3a, g3b, dinv, b3, lw1p)
    gath = _link_gather_call(pq.reshape(2 * npad, -1), lidx)
    return _tc_link_call(gath, lb1p, lw2p, LB2, ew)


# speedup vs baseline: 20.8810x; 1.0546x over previous
import functools

import jax
import jax.numpy as jnp
from jax import lax
from jax.experimental import pallas as pl
from jax.experimental.pallas import tpu as pltpu
from jax.experimental.pallas import tpu_sc as plsc

_LANES = 128
_CP = 56
_BN = 3584
_BE = 6400


def _sc_mesh():
    return plsc.VectorSubcoreMesh(core_axis_name="c", subcore_axis_name="s")


def _deg_call(dst_rows, zeros1):
    npad = zeros1.shape[0]
    epr = dst_rows.shape[0]
    per_sc = epr // 2
    per_tile = per_sc // 16
    rows_out = npad // 16

    @functools.partial(
        pl.kernel,
        out_type=jax.ShapeDtypeStruct((2 * npad,), jnp.float32),
        mesh=_sc_mesh(),
        compiler_params=pltpu.CompilerParams(use_tc_tiling_on_sc=False),
        scratch_types=[
            pltpu.VMEM((per_tile, _LANES), jnp.int32),
            pltpu.VMEM((_LANES,), jnp.float32),
            pltpu.VMEM_SHARED((npad,), jnp.float32),
            pltpu.SemaphoreType.DMA,
        ],
    )
    def k(dst_hbm, z_hbm, out_hbm, idx_v, ones_v, acc, sem):
        c = lax.axis_index("c")
        s = lax.axis_index("s")
        for i in range(_LANES // 16):
            ones_v[pl.ds(16 * i, 16)] = jnp.ones((16,), jnp.float32)
        pltpu.sync_copy(z_hbm.at[pl.ds(s * rows_out, rows_out)],
                        acc.at[pl.ds(s * rows_out, rows_out)])
        plsc.subcore_barrier()
        base = c * per_sc + s * per_tile
        pltpu.sync_copy(dst_hbm.at[pl.ds(base, per_tile)], idx_v)

        def group(m, cc):
            for t in range(8):
                pltpu.async_copy(
                    ones_v, acc.at[idx_v.at[8 * m + t]], sem, add=True)
            for t in range(8):
                pltpu.make_async_copy(
                    ones_v, acc.at[idx_v.at[0]], sem).wait()
            return cc

        lax.fori_loop(0, per_tile // 8, group, 0)
        plsc.subcore_barrier()
        pltpu.sync_copy(acc.at[pl.ds(s * rows_out, rows_out)],
                        out_hbm.at[pl.ds(c * npad + s * rows_out, rows_out)])

    return k(dst_rows, zeros1)


def _agg_pipeline(g_hbm, src_hbm, dst_hbm, acc, idxs_v, idxd_v,
                  bufs, gsems, ssems, base, phases):
    nb = len(bufs)
    for ph in range(phases):
        pb = base + ph * _CP
        pltpu.sync_copy(src_hbm.at[pl.ds(pb, _CP)], idxs_v)
        pltpu.sync_copy(dst_hbm.at[pl.ds(pb, _CP)], idxd_v)
        for t in range(nb):
            pltpu.async_copy(g_hbm.at[idxs_v.at[t]], bufs[t], gsems[t])

        def group(m, cc):
            j0 = nb * m
            for t in range(nb):
                pltpu.make_async_copy(
                    g_hbm.at[idxs_v.at[0]], bufs[t], gsems[t]).wait()
                pltpu.async_copy(
                    bufs[t], acc.at[idxd_v.at[j0 + t]], ssems[t], add=True)

            @pl.when(m < _CP // nb - 1)
            def _():
                for t in range(nb):
                    pltpu.make_async_copy(
                        bufs[t], acc.at[idxd_v.at[0]], ssems[t]).wait()
                    pltpu.async_copy(
                        g_hbm.at[idxs_v.at[j0 + nb + t]], bufs[t], gsems[t])
            return cc

        lax.fori_loop(0, _CP // nb, group, 0)
        for t in range(nb):
            pltpu.make_async_copy(
                bufs[t], acc.at[idxd_v.at[0]], ssems[t]).wait()


def _agg_edge_call(g, src_rows, dst_rows, zeros_d):
    npad, d = g.shape
    epr = src_rows.shape[0]
    per_sc = epr // 2
    per_tile = per_sc // 16
    phases = per_tile // _CP
    assert per_tile == phases * _CP
    rows_out = npad // 16

    @functools.partial(
        pl.kernel,
        out_type=jax.ShapeDtypeStruct((2, npad, d), jnp.float32),
        mesh=_sc_mesh(),
        compiler_params=pltpu.CompilerParams(use_tc_tiling_on_sc=False),
        scratch_types=[
            pltpu.VMEM((_CP, _LANES), jnp.int32),
            pltpu.VMEM((_CP, _LANES), jnp.int32),
            [pltpu.VMEM((_LANES, d), jnp.float32)] * 8,
            pltpu.VMEM_SHARED((npad, d), jnp.float32),
            [pltpu.SemaphoreType.DMA] * 8,
            [pltpu.SemaphoreType.DMA] * 8,
        ],
    )
    def k(g_hbm, src_hbm, dst_hbm, z_hbm, out_hbm,
          idxs_v, idxd_v, bufs, acc, gsems, ssems):
        c = lax.axis_index("c")
        s = lax.axis_index("s")
        pltpu.sync_copy(z_hbm.at[pl.ds(s * rows_out, rows_out)],
                        acc.at[pl.ds(s * rows_out, rows_out)])
        plsc.subcore_barrier()
        base = c * per_sc + s * per_tile
        _agg_pipeline(g_hbm, src_hbm, dst_hbm, acc, idxs_v, idxd_v,
                      bufs, gsems, ssems, base, phases=phases)
        plsc.subcore_barrier()
        pltpu.sync_copy(acc.at[pl.ds(s * rows_out, rows_out)],
                        out_hbm.at[c, pl.ds(s * rows_out, rows_out)])

    return k(g, src_rows, dst_rows, zeros_d)


def _agg_col_call(ga, gb, src_rows, dst_rows, zeros_d):
    npad, d = ga.shape
    epr = src_rows.shape[0]
    per_tile = epr // 16
    phases = per_tile // _CP
    assert per_tile == phases * _CP
    rows_out = npad // 16

    @functools.partial(
        pl.kernel,
        out_type=jax.ShapeDtypeStruct((2, npad, d), jnp.float32),
        mesh=_sc_mesh(),
        compiler_params=pltpu.CompilerParams(use_tc_tiling_on_sc=False),
        scratch_types=[
            pltpu.VMEM((_CP, _LANES), jnp.int32),
            pltpu.VMEM((_CP, _LANES), jnp.int32),
            [pltpu.VMEM((_LANES, d), jnp.float32)] * 8,
            pltpu.VMEM_SHARED((npad, d), jnp.float32),
            [pltpu.SemaphoreType.DMA] * 8,
            [pltpu.SemaphoreType.DMA] * 8,
        ],
    )
    def k(ga_hbm, gb_hbm, src_hbm, dst_hbm, z_hbm, out_hbm,
          idxs_v, idxd_v, bufs, acc, gsems, ssems):
        c = lax.axis_index("c")
        s = lax.axis_index("s")
        pltpu.sync_copy(z_hbm.at[pl.ds(s * rows_out, rows_out)],
                        acc.at[pl.ds(s * rows_out, rows_out)])
        plsc.subcore_barrier()
        base = s * per_tile

        @pl.when(c == 0)
        def _():
            _agg_pipeline(ga_hbm, src_hbm, dst_hbm, acc, idxs_v, idxd_v,
                          bufs, gsems, ssems, base, phases=phases)

        @pl.when(c != 0)
        def _():
            _agg_pipeline(gb_hbm, src_hbm, dst_hbm, acc, idxs_v, idxd_v,
                          bufs, gsems, ssems, base, phases=phases)

        plsc.subcore_barrier()
        pltpu.sync_copy(acc.at[pl.ds(s * rows_out, rows_out)],
                        out_hbm.at[c, pl.ds(s * rows_out, rows_out)])

    return k(ga, gb, src_rows, dst_rows, zeros_d)


def _link_gather_call(tbl, idx_rows):
    _, d = tbl.shape
    lpr = idx_rows.shape[0]
    per_tile = lpr // 32

    @functools.partial(
        pl.kernel,
        out_type=jax.ShapeDtypeStruct((lpr * _LANES, d), jnp.float32),
        mesh=_sc_mesh(),
        compiler_params=pltpu.CompilerParams(use_tc_tiling_on_sc=False),
        scratch_types=[
            pltpu.VMEM((per_tile, _LANES), jnp.int32),
            [pltpu.VMEM((_LANES, d), jnp.float32)] * 8,
            [pltpu.SemaphoreType.DMA] * 8,
            [pltpu.SemaphoreType.DMA] * 8,
        ],
    )
    def k(t_hbm, idx_hbm, out_hbm, idx_v, bufs, gsems, ssems):
        c = lax.axis_index("c")
        s = lax.axis_index("s")
        base = (s * 2 + c) * per_tile
        nb = len(bufs)
        pltpu.sync_copy(idx_hbm.at[pl.ds(base, per_tile)], idx_v)
        for t in range(nb):
            pltpu.async_copy(t_hbm.at[idx_v.at[t]], bufs[t], gsems[t])

        def group(m, cc):
            j0 = nb * m
            for t in range(nb):
                pltpu.make_async_copy(
                    t_hbm.at[idx_v.at[0]], bufs[t], gsems[t]).wait()
                pltpu.async_copy(
                    bufs[t],
                    out_hbm.at[pl.ds((base + j0 + t) * _LANES, _LANES)],
                    ssems[t])

            @pl.when(m < per_tile // nb - 1)
            def _():
                for t in range(nb):
                    pltpu.make_async_copy(
                        bufs[t], out_hbm.at[pl.ds(0, _LANES)],
                        ssems[t]).wait()
                    pltpu.async_copy(
                        t_hbm.at[idx_v.at[j0 + nb + t]], bufs[t],
                        gsems[t])
            return cc

        lax.fori_loop(0, per_tile // nb, group, 0)
        for t in range(nb):
            pltpu.make_async_copy(
                bufs[t], out_hbm.at[pl.ds(0, _LANES)], ssems[t]).wait()

    return k(tbl, idx_rows)


def _tc_first_call(deg2, xp, w1):
    npad, f = xp.shape
    dn = w1.shape[1]
    grid = npad // _BN

    def body(d_ref, x_ref, w_ref, dinv_ref, g_ref):
        deg = d_ref[0] + d_ref[1] + 1.0
        dinv = lax.rsqrt(deg)
        h = jnp.dot(x_ref[...], w_ref[...], preferred_element_type=jnp.float32)
        dinv_ref[...] = dinv[:, None]
        g_ref[...] = h * dinv[:, None]

    return pl.pallas_call(
        body,
        grid=(grid,),
        in_specs=[pl.BlockSpec((2, _BN), lambda i: (0, i)),
                  pl.BlockSpec((_BN, f), lambda i: (i, 0)),
                  pl.BlockSpec(w1.shape, lambda i: (0, 0))],
        out_specs=[pl.BlockSpec((_BN, 1), lambda i: (i, 0)),
                   pl.BlockSpec((_BN, dn), lambda i: (i, 0))],
        out_shape=[jax.ShapeDtypeStruct((npad, 1), jnp.float32),
                   jax.ShapeDtypeStruct((npad, dn), jnp.float32)],
    )(deg2, xp, w1)


def _tc_layer_call(acc, gprev, dinv, b, w, split):
    _, npad, di = acc.shape
    dn = w.shape[1]
    grid = npad // _BN

    def body(a_ref, g_ref, di_ref, b_ref, w_ref, *outs):
        dv = di_ref[...]
        h = jax.nn.relu(dv * (a_ref[0] + a_ref[1] + g_ref[...])
                        + b_ref[...])
        gn = jnp.dot(h, w_ref[...],
                     preferred_element_type=jnp.float32) * dv
        if split:
            outs[0][...] = gn[:, :dn // 2]
            outs[1][...] = gn[:, dn // 2:]
        else:
            outs[0][...] = gn

    if split:
        out_specs = [pl.BlockSpec((_BN, dn // 2), lambda i: (i, 0)),
                     pl.BlockSpec((_BN, dn // 2), lambda i: (i, 0))]
        out_shape = [jax.ShapeDtypeStruct((npad, dn // 2), jnp.float32),
                     jax.ShapeDtypeStruct((npad, dn // 2), jnp.float32)]
    else:
        out_specs = [pl.BlockSpec((_BN, dn), lambda i: (i, 0))]
        out_shape = [jax.ShapeDtypeStruct((npad, dn), jnp.float32)]

    return pl.pallas_call(
        body,
        grid=(grid,),
        in_specs=[pl.BlockSpec((2, _BN, di), lambda i: (0, i, 0)),
                  pl.BlockSpec((_BN, di), lambda i: (i, 0)),
                  pl.BlockSpec((_BN, 1), lambda i: (i, 0)),
                  pl.BlockSpec(b.shape, lambda i: (0,)),
                  pl.BlockSpec(w.shape, lambda i: (0, 0))],
        out_specs=out_specs,
        out_shape=out_shape,
    )(acc, gprev, dinv, b, w)


def _tc_pq_call(acc3, g3a, g3b, dinv, b3, lw1):
    _, npad, dh = acc3.shape
    d4 = lw1.shape[1]
    grid = npad // _BN

    def body(a_ref, ga_ref, gb_ref, di_ref, b_ref, w_ref, o_ref):
        dv = di_ref[...]
        bb = b_ref[...]
        w = w_ref[...]
        ha = jax.nn.relu(dv * (a_ref[0] + ga_ref[...]) + bb[:dh])
        hb = jax.nn.relu(dv * (a_ref[1] + gb_ref[...]) + bb[dh:])
        p = (jnp.dot(ha, w[:dh], preferred_element_type=jnp.float32)
             + jnp.dot(hb, w[dh:2 * dh], preferred_element_type=jnp.float32))
        q = (jnp.dot(ha, w[2 * dh:3 * dh], preferred_element_type=jnp.float32)
             + jnp.dot(hb, w[3 * dh:], preferred_element_type=jnp.float32))
        o_ref[0] = p
        o_ref[1] = q

    return pl.pallas_call(
        body,
        grid=(grid,),
        in_specs=[pl.BlockSpec((2, _BN, dh), lambda i: (0, i, 0)),
                  pl.BlockSpec((_BN, dh), lambda i: (i, 0)),
                  pl.BlockSpec((_BN, dh), lambda i: (i, 0)),
                  pl.BlockSpec((_BN, 1), lambda i: (i, 0)),
                  pl.BlockSpec((2 * dh,), lambda i: (0,)),
                  pl.BlockSpec(lw1.shape, lambda i: (0, 0))],
        out_specs=pl.BlockSpec((2, _BN, d4), lambda i: (0, i, 0)),
        out_shape=jax.ShapeDtypeStruct((2, npad, d4), jnp.float32),
    )(acc3, g3a, g3b, dinv, b3, lw1)


def _tc_link_call(gath, lb1, lw2, lb2, ew):
    d4 = gath.shape[1]
    do = lw2.shape[1]
    grid = ew // _BE

    def body(a_ref, b_ref, l1, w2, l2, o_ref):
        hidden = a_ref[...] + b_ref[...] + l1[...]
        o_ref[...] = jnp.dot(hidden, w2[...],
                             preferred_element_type=jnp.float32) + l2[...]

    return pl.pallas_call(
        body,
        grid=(grid,),
        in_specs=[pl.BlockSpec((_BE, d4), lambda i: (i, 0)),
                  pl.BlockSpec((_BE, d4), lambda i, g=grid: (i + g, 0)),
                  pl.BlockSpec((d4,), lambda i: (0,)),
                  pl.BlockSpec((d4, do), lambda i: (0, 0)),
                  pl.BlockSpec((do,), lambda i: (0,))],
        out_specs=pl.BlockSpec((_BE, do), lambda i: (i, 0)),
        out_shape=jax.ShapeDtypeStruct((ew, do), jnp.float32),
    )(gath, gath, lb1, lw2, lb2)


def kernel(x, edge_index, w2b, W1, b1, W2, b2, W3, b3, LW1, LB1, LW2, LB2):
    n, _ = x.shape
    e = edge_index.shape[1]
    ew = w2b.shape[1]

    npad = ((n + 1 + _BN - 1) // _BN) * _BN
    eq = 2 * 16 * _CP * _LANES
    epad = ((e + eq - 1) // eq) * eq
    lpad = ((2 * ew + eq - 1) // eq) * eq

    idx_pad = jnp.full((epad - e,), n, jnp.int32)
    srcp = jnp.concatenate([edge_index[0], idx_pad]).reshape(-1, _LANES)
    dstp = jnp.concatenate([edge_index[1], idx_pad]).reshape(-1, _LANES)
    lidx = jnp.concatenate(
        [w2b[0], w2b[1] + npad,
         jnp.full((lpad - 2 * ew,), n, jnp.int32)]).reshape(-1, _LANES)
    xp = jnp.pad(x, ((0, npad - n), (0, 0)))

    z1 = jnp.zeros((npad,), jnp.float32)
    zA = jnp.zeros((npad, W1.shape[1]), jnp.float32)
    zB = jnp.zeros((npad, W2.shape[1]), jnp.float32)

    deg2 = _deg_call(dstp, z1).reshape(2, npad)
    dinv, g1 = _tc_first_call(deg2, xp, W1)
    acc1 = _agg_edge_call(g1, srcp, dstp, zA)
    (g2,) = _tc_layer_call(acc1, g1, dinv, b1, W2, split=False)
    acc2 = _agg_edge_call(g2, srcp, dstp, zB)
    g3a, g3b = _tc_layer_call(acc2, g2, dinv, b2, W3, split=True)
    acc3 = _agg_col_call(g3a, g3b, srcp, dstp, zB)
    dp = 16 - LW1.shape[1]
    lw1p = jnp.pad(LW1, ((0, 0), (0, dp)))
    lb1p = jnp.pad(LB1, (0, dp))
    lw2p = jnp.pad(LW2, ((0, dp), (0, 0)))
    pq = _tc_pq_call(acc3, g3a, g3b, dinv, b3, lw1p)
    gath = _link_gather_call(pq.reshape(2 * npad, -1), lidx)
    return _tc_link_call(gath, lb1p, lw2p, LB2, ew)
